# baseline (speedup 1.0000x reference)
import jax
import jax.numpy as jnp
from jax.experimental import pallas as pl
from jax.experimental.pallas import tpu as pltpu

_OBJECT_SCALE = 5.0
_NOOBJECT_SCALE = 1.0
_IGNORE_THRES = 0.6


def _slog(s):
    return jnp.max(jnp.log(jnp.broadcast_to(s, (8, 128))))


def _sexp(s):
    return jnp.max(jnp.exp(jnp.broadcast_to(s, (8, 128))))


def _region_body(nB, nA, nH, nW):
    def body(out_ref, tgt_ref, anc_ref, loss_ref, r50_ref, r75_ref, aiou_ref, acc_ref):
        b = pl.program_id(0)

        t0 = tgt_ref[b, 0]
        t1 = tgt_ref[b, 1]
        t2 = tgt_ref[b, 2]
        t3 = tgt_ref[b, 3]
        gt_x = t0 * nW
        gt_y = t1 * nH
        gt_w = t2 * nW
        gt_h = t3 * nH
        gxf = jnp.floor(gt_x)
        gyf = jnp.floor(gt_y)
        gx = gxf.astype(jnp.int32)
        gy = gyf.astype(jnp.int32)

        aw = [anc_ref[a, 0] for a in range(nA)]
        ah = [anc_ref[a, 1] for a in range(nA)]

        ratios = []
        for a in range(nA):
            inter = jnp.minimum(gt_w, aw[a]) * jnp.minimum(gt_h, ah[a])
            union = gt_w * gt_h + 1e-16 + aw[a] * ah[a] - inter
            ratios.append(inter / union)
        best = ratios[0]
        for a in range(1, nA):
            best = jnp.maximum(best, ratios[a])
        sels = []
        found = ratios[0] < ratios[0]
        for a in range(nA):
            is_best = jnp.logical_and(ratios[a] >= best, jnp.logical_not(found))
            sels.append(is_best)
            found = jnp.logical_or(found, is_best)

        b2x1 = gt_x - gt_w / 2
        b2x2 = gt_x + gt_w / 2
        b2y1 = gt_y - gt_h / 2
        b2y2 = gt_y + gt_h / 2
        a2 = (b2x2 - b2x1 + 1.0) * (b2y2 - b2y1 + 1.0)

        rowf = jax.lax.broadcasted_iota(jnp.int32, (nH, nW), 0).astype(jnp.float32)
        colf = jax.lax.broadcasted_iota(jnp.int32, (nH, nW), 1).astype(jnp.float32)

        n2_map = jnp.zeros((nH, nW), jnp.float32)
        cnt_map = jnp.zeros((nH, nW), jnp.float32)
        raw = [jnp.float32(0.0)] * 5

        lane = jax.lax.broadcasted_iota(jnp.int32, (1, nW), 1)
        colmask = lane == gx

        for a in range(nA):
            base = 5 * a
            xo = out_ref[0, base + 0]
            yo = out_ref[0, base + 1]
            wo = out_ref[0, base + 2]
            ho = out_ref[0, base + 3]
            co = out_ref[0, base + 4]
            x = 1.0 / (1.0 + jnp.exp(-xo))
            y = 1.0 / (1.0 + jnp.exp(-yo))
            conf = 1.0 / (1.0 + jnp.exp(-co))
            px = x + colf
            py = y + rowf
            pw = jnp.exp(wo) * aw[a]
            ph = jnp.exp(ho) * ah[a]

            hw = pw * 0.5
            hh = ph * 0.5
            b1x1 = px - hw
            b1x2 = px + hw
            b1y1 = py - hh
            b1y2 = py + hh
            ix1 = jnp.maximum(b1x1, b2x1)
            iy1 = jnp.maximum(b1y1, b2y1)
            ix2 = jnp.minimum(b1x2, b2x2)
            iy2 = jnp.minimum(b1y2, b2y2)
            inter = jnp.maximum(ix2 - ix1 + 1.0, 0.0) * jnp.maximum(iy2 - iy1 + 1.0, 0.0)
            a1 = (b1x2 - b1x1 + 1.0) * (b1y2 - b1y1 + 1.0)
            noobj = inter <= _IGNORE_THRES * (a1 + a2 - inter + 1e-16)
            cm = jnp.where(noobj, conf, 0.0)
            n2_map = n2_map + cm * cm
            cnt_map = cnt_map + jnp.where(noobj, 1.0, 0.0)

            for c in range(5):
                rv = out_ref[0, base + c, pl.ds(gy, 1), :]
                v = jnp.sum(jnp.where(colmask, rv, 0.0))
                raw[c] = raw[c] + jnp.where(sels[a], v, 0.0)

        s_n2 = jnp.sum(n2_map)
        s_cnt = jnp.sum(cnt_map)

        a_w_best = 0.0
        a_h_best = 0.0
        for a in range(nA):
            a_w_best = a_w_best + jnp.where(sels[a], aw[a], 0.0)
            a_h_best = a_h_best + jnp.where(sels[a], ah[a], 0.0)

        x_obj = 1.0 / (1.0 + _sexp(-raw[0]))
        y_obj = 1.0 / (1.0 + _sexp(-raw[1]))
        conf_obj = 1.0 / (1.0 + _sexp(-raw[4]))
        pw_obj = _sexp(raw[2]) * a_w_best
        ph_obj = _sexp(raw[3]) * a_h_best

        tx = gt_x - gxf
        ty = gt_y - gyf
        tw = _slog(gt_w / a_w_best + 1e-16)
        th = _slog(gt_h / a_h_best + 1e-16)
        scale = 2.0 - t2 * t3

        sq_x = (x_obj * scale - tx * scale) ** 2
        sq_y = (y_obj * scale - ty * scale) ** 2
        sq_w = (raw[2] * scale - tw * scale) ** 2
        sq_h = (raw[3] * scale - th * scale) ** 2
        sq_conf = (conf_obj - 1.0) ** 2

        px_o = x_obj + gxf
        py_o = y_obj + gyf
        hwo = pw_obj * 0.5
        hho = ph_obj * 0.5
        p1x1 = px_o - hwo
        p1x2 = px_o + hwo
        p1y1 = py_o - hho
        p1y2 = py_o + hho
        jx1 = jnp.maximum(p1x1, b2x1)
        jy1 = jnp.maximum(p1y1, b2y1)
        jx2 = jnp.minimum(p1x2, b2x2)
        jy2 = jnp.minimum(p1y2, b2y2)
        jinter = jnp.maximum(jx2 - jx1 + 1.0, 0.0) * jnp.maximum(jy2 - jy1 + 1.0, 0.0)
        ja1 = (p1x2 - p1x1 + 1.0) * (p1y2 - p1y1 + 1.0)
        jt = ja1 + a2 - jinter + 1e-16
        iou_v = jinter / jt

        incl = jinter <= _IGNORE_THRES * jt
        s_n2 = s_n2 - jnp.where(incl, conf_obj * conf_obj, 0.0)
        s_cnt = s_cnt - jnp.where(incl, 1.0, 0.0)

        @pl.when(b == 0)
        def _init():
            for i in range(10):
                acc_ref[i] = 0.0

        vals = [
            sq_x, sq_y, sq_w, sq_h, sq_conf, s_n2, s_cnt,
            jnp.where(iou_v > 0.5, 1.0, 0.0),
            jnp.where(iou_v > 0.75, 1.0, 0.0),
            iou_v,
        ]
        for i, v in enumerate(vals):
            acc_ref[i] = acc_ref[i] + v

        @pl.when(b == nB - 1)
        def _fin():
            fnB = float(nB)
            n_noobj = jnp.maximum(acc_ref[6], 1.0)
            loss = (acc_ref[0] + acc_ref[1] + acc_ref[2] + acc_ref[3]
                    + _OBJECT_SCALE * acc_ref[4]) / fnB \
                + _NOOBJECT_SCALE * acc_ref[5] / n_noobj
            loss_ref[0] = loss
            r50_ref[0] = acc_ref[7] / fnB
            r75_ref[0] = acc_ref[8] / fnB
            aiou_ref[0] = acc_ref[9] / fnB

    return body


def kernel(output, targets, anchors):
    nB, C, nH, nW = output.shape
    nA = anchors.shape[0]
    body = _region_body(nB, nA, nH, nW)
    outs = pl.pallas_call(
        body,
        grid=(nB,),
        in_specs=[
            pl.BlockSpec((1, C, nH, nW), lambda b: (b, 0, 0, 0)),
            pl.BlockSpec(memory_space=pltpu.SMEM),
            pl.BlockSpec(memory_space=pltpu.SMEM),
        ],
        out_specs=[pl.BlockSpec(memory_space=pltpu.SMEM)] * 4,
        out_shape=[jax.ShapeDtypeStruct((1,), jnp.float32)] * 4,
        scratch_shapes=[pltpu.SMEM((10,), jnp.float32)],
    )(output, targets, anchors)
    return tuple(o[0] for o in outs)


# speedup vs baseline: 1.3838x; 1.3838x over previous
import jax
import jax.numpy as jnp
from jax.experimental import pallas as pl
from jax.experimental.pallas import tpu as pltpu

_OBJECT_SCALE = 5.0
_NOOBJECT_SCALE = 1.0
_IGNORE_THRES = 0.6


def _slog(s):
    return jnp.max(jnp.log(jnp.broadcast_to(s, (8, 128))))


def _sexp(s):
    return jnp.max(jnp.exp(jnp.broadcast_to(s, (8, 128))))


def _region_body(nB, nA, nH, nW, G):
    nsteps = nB // G

    def body(out_ref, tgt_ref, anc_ref, loss_ref, r50_ref, r75_ref, aiou_ref, acc_ref):
        s = pl.program_id(0)

        @pl.when(s == 0)
        def _init():
            for i in range(10):
                acc_ref[i] = 0.0

        n2_map = jnp.zeros((nH, nW), jnp.float32)
        cnt_map = jnp.zeros((nH, nW), jnp.float32)
        rowf = jax.lax.broadcasted_iota(jnp.int32, (nH, nW), 0).astype(jnp.float32)
        colf = jax.lax.broadcasted_iota(jnp.int32, (nH, nW), 1).astype(jnp.float32)
        lane = jax.lax.broadcasted_iota(jnp.int32, (1, nW), 1)

        accs = [0.0] * 10
        for g in range(G):
            _one_batch(out_ref, tgt_ref, anc_ref, s * G + g, g, nA, nH, nW,
                       rowf, colf, lane, n2_map, cnt_map, accs)
            n2_map, cnt_map = accs[10], accs[11]
            del accs[10:]

        s_n2 = jnp.sum(n2_map) + accs[5]
        s_cnt = jnp.sum(cnt_map) + accs[6]
        accs[5] = s_n2
        accs[6] = s_cnt

        for i, v in enumerate(accs):
            acc_ref[i] = acc_ref[i] + v

        @pl.when(s == nsteps - 1)
        def _fin():
            fnB = float(nB)
            n_noobj = jnp.maximum(acc_ref[6], 1.0)
            loss = (acc_ref[0] + acc_ref[1] + acc_ref[2] + acc_ref[3]
                    + _OBJECT_SCALE * acc_ref[4]) / fnB \
                + _NOOBJECT_SCALE * acc_ref[5] / n_noobj
            loss_ref[0] = loss
            r50_ref[0] = acc_ref[7] / fnB
            r75_ref[0] = acc_ref[8] / fnB
            aiou_ref[0] = acc_ref[9] / fnB

    return body


def _one_batch(out_ref, tgt_ref, anc_ref, b, g, nA, nH, nW,
               rowf, colf, lane, n2_map, cnt_map, accs):
        t0 = tgt_ref[b, 0]
        t1 = tgt_ref[b, 1]
        t2 = tgt_ref[b, 2]
        t3 = tgt_ref[b, 3]
        gt_x = t0 * nW
        gt_y = t1 * nH
        gt_w = t2 * nW
        gt_h = t3 * nH
        gxf = jnp.floor(gt_x)
        gyf = jnp.floor(gt_y)
        gx = gxf.astype(jnp.int32)
        gy = gyf.astype(jnp.int32)

        aw = [anc_ref[a, 0] for a in range(nA)]
        ah = [anc_ref[a, 1] for a in range(nA)]

        ratios = []
        for a in range(nA):
            inter = jnp.minimum(gt_w, aw[a]) * jnp.minimum(gt_h, ah[a])
            union = gt_w * gt_h + 1e-16 + aw[a] * ah[a] - inter
            ratios.append(inter / union)
        best = ratios[0]
        for a in range(1, nA):
            best = jnp.maximum(best, ratios[a])
        sels = []
        found = ratios[0] < ratios[0]
        for a in range(nA):
            is_best = jnp.logical_and(ratios[a] >= best, jnp.logical_not(found))
            sels.append(is_best)
            found = jnp.logical_or(found, is_best)

        b2x1 = gt_x - gt_w / 2
        b2x2 = gt_x + gt_w / 2
        b2y1 = gt_y - gt_h / 2
        b2y2 = gt_y + gt_h / 2
        a2 = (b2x2 - b2x1 + 1.0) * (b2y2 - b2y1 + 1.0)

        raw = [jnp.float32(0.0)] * 5
        colmask = lane == gx

        for a in range(nA):
            base = 5 * a
            xo = out_ref[g, base + 0]
            yo = out_ref[g, base + 1]
            wo = out_ref[g, base + 2]
            ho = out_ref[g, base + 3]
            co = out_ref[g, base + 4]
            x = 1.0 / (1.0 + jnp.exp(-xo))
            y = 1.0 / (1.0 + jnp.exp(-yo))
            conf = 1.0 / (1.0 + jnp.exp(-co))
            px = x + colf
            py = y + rowf
            pw = jnp.exp(wo) * aw[a]
            ph = jnp.exp(ho) * ah[a]

            hw = pw * 0.5
            hh = ph * 0.5
            b1x1 = px - hw
            b1x2 = px + hw
            b1y1 = py - hh
            b1y2 = py + hh
            ix1 = jnp.maximum(b1x1, b2x1)
            iy1 = jnp.maximum(b1y1, b2y1)
            ix2 = jnp.minimum(b1x2, b2x2)
            iy2 = jnp.minimum(b1y2, b2y2)
            inter = jnp.maximum(ix2 - ix1 + 1.0, 0.0) * jnp.maximum(iy2 - iy1 + 1.0, 0.0)
            a1 = (b1x2 - b1x1 + 1.0) * (b1y2 - b1y1 + 1.0)
            noobj = inter <= _IGNORE_THRES * (a1 + a2 - inter + 1e-16)
            cm = jnp.where(noobj, conf, 0.0)
            n2_map = n2_map + cm * cm
            cnt_map = cnt_map + jnp.where(noobj, 1.0, 0.0)

            for c in range(5):
                rv = out_ref[g, base + c, pl.ds(gy, 1), :]
                v = jnp.sum(jnp.where(colmask, rv, 0.0))
                raw[c] = raw[c] + jnp.where(sels[a], v, 0.0)

        a_w_best = 0.0
        a_h_best = 0.0
        for a in range(nA):
            a_w_best = a_w_best + jnp.where(sels[a], aw[a], 0.0)
            a_h_best = a_h_best + jnp.where(sels[a], ah[a], 0.0)

        x_obj = 1.0 / (1.0 + _sexp(-raw[0]))
        y_obj = 1.0 / (1.0 + _sexp(-raw[1]))
        conf_obj = 1.0 / (1.0 + _sexp(-raw[4]))
        pw_obj = _sexp(raw[2]) * a_w_best
        ph_obj = _sexp(raw[3]) * a_h_best

        tx = gt_x - gxf
        ty = gt_y - gyf
        tw = _slog(gt_w / a_w_best + 1e-16)
        th = _slog(gt_h / a_h_best + 1e-16)
        scale = 2.0 - t2 * t3

        sq_x = (x_obj * scale - tx * scale) ** 2
        sq_y = (y_obj * scale - ty * scale) ** 2
        sq_w = (raw[2] * scale - tw * scale) ** 2
        sq_h = (raw[3] * scale - th * scale) ** 2
        sq_conf = (conf_obj - 1.0) ** 2

        px_o = x_obj + gxf
        py_o = y_obj + gyf
        hwo = pw_obj * 0.5
        hho = ph_obj * 0.5
        p1x1 = px_o - hwo
        p1x2 = px_o + hwo
        p1y1 = py_o - hho
        p1y2 = py_o + hho
        jx1 = jnp.maximum(p1x1, b2x1)
        jy1 = jnp.maximum(p1y1, b2y1)
        jx2 = jnp.minimum(p1x2, b2x2)
        jy2 = jnp.minimum(p1y2, b2y2)
        jinter = jnp.maximum(jx2 - jx1 + 1.0, 0.0) * jnp.maximum(jy2 - jy1 + 1.0, 0.0)
        ja1 = (p1x2 - p1x1 + 1.0) * (p1y2 - p1y1 + 1.0)
        jt = ja1 + a2 - jinter + 1e-16
        iou_v = jinter / jt

        incl = jinter <= _IGNORE_THRES * jt
        corr_n2 = -jnp.where(incl, conf_obj * conf_obj, 0.0)
        corr_cnt = -jnp.where(incl, 1.0, 0.0)

        vals = [
            sq_x, sq_y, sq_w, sq_h, sq_conf, corr_n2, corr_cnt,
            jnp.where(iou_v > 0.5, 1.0, 0.0),
            jnp.where(iou_v > 0.75, 1.0, 0.0),
            iou_v,
        ]
        for i, v in enumerate(vals):
            accs[i] = accs[i] + v
        accs.append(n2_map)
        accs.append(cnt_map)


def kernel(output, targets, anchors):
    nB, C, nH, nW = output.shape
    nA = anchors.shape[0]
    G = 8
    body = _region_body(nB, nA, nH, nW, G)
    outs = pl.pallas_call(
        body,
        grid=(nB // G,),
        in_specs=[
            pl.BlockSpec((G, C, nH, nW), lambda b: (b, 0, 0, 0)),
            pl.BlockSpec(memory_space=pltpu.SMEM),
            pl.BlockSpec(memory_space=pltpu.SMEM),
        ],
        out_specs=[pl.BlockSpec(memory_space=pltpu.SMEM)] * 4,
        out_shape=[jax.ShapeDtypeStruct((1,), jnp.float32)] * 4,
        scratch_shapes=[pltpu.SMEM((10,), jnp.float32)],
    )(output, targets, anchors)
    return tuple(o[0] for o in outs)
